# Initial kernel scaffold; baseline (speedup 1.0000x reference)
import jax
import jax.numpy as jnp
from jax.experimental import pallas as pl


def kernel(x, y):
    raise NotImplementedError("write your pallas kernel here")


# speedup vs baseline: 24.1456x; 24.1456x over previous
import jax
import jax.numpy as jnp
from jax.experimental import pallas as pl
from jax.experimental.pallas import tpu as pltpu

_POS_TH = 2.0
_N = 512 * 512
_INF_BITS = 0x7F800000


def _body(x_ref, y_ref, out_ref, nb_ref):
    xv = x_ref[0]
    yv = y_ref[0]
    d = xv - yv
    loss = d * d
    pos = yv > _POS_TH
    p = jnp.sum(pos.astype(jnp.int32))
    pos_sum = jnp.sum(jnp.where(pos, loss, 0.0))
    total = jnp.sum(loss)
    bits = jax.lax.bitcast_convert_type(loss, jnp.int32)
    nb_ref[...] = jnp.where(pos, 0, bits)
    k = 3 * p

    def srch(_, carry):
        lo, hi = carry
        mid = lo + (hi - lo) // 2
        cnt = jnp.sum((nb_ref[...] >= mid).astype(jnp.int32))
        ok = cnt >= k
        return jnp.where(ok, mid, lo), jnp.where(ok, hi, mid)

    t_bits, _ = jax.lax.fori_loop(
        0, 31, srch, (jnp.int32(0), jnp.int32(_INF_BITS))
    )

    nb = nb_ref[...]
    gt = nb >= (t_bits + 1)
    cnt_gt = jnp.sum(gt.astype(jnp.int32))
    vals = jax.lax.bitcast_convert_type(nb, jnp.float32)
    sum_gt = jnp.sum(jnp.where(gt, vals, 0.0))
    t = jax.lax.bitcast_convert_type(t_bits, jnp.float32)

    kf = k.astype(jnp.float32)
    pf = p.astype(jnp.float32)
    topk = sum_gt + (kf - cnt_gt.astype(jnp.float32)) * t
    fallback = total / _N
    mined = pos_sum / jnp.maximum(pf, 1.0) + topk / jnp.maximum(kf, 1.0)
    cond = (k + p >= _N) | (k <= 10)
    out_ref[...] = jnp.broadcast_to(jnp.where(cond, fallback, mined), (1, 1, 128))


def kernel(x, y):
    B = x.shape[0]
    xs = x.reshape(B, 512, 512)
    ys = y.reshape(B, 512, 512)
    out = pl.pallas_call(
        _body,
        grid=(B,),
        in_specs=[
            pl.BlockSpec((1, 512, 512), lambda i: (i, 0, 0)),
            pl.BlockSpec((1, 512, 512), lambda i: (i, 0, 0)),
        ],
        out_specs=pl.BlockSpec((1, 1, 128), lambda i: (i, 0, 0)),
        out_shape=jax.ShapeDtypeStruct((B, 1, 128), jnp.float32),
        scratch_shapes=[pltpu.VMEM((512, 512), jnp.int32)],
    )(xs, ys)
    return jnp.mean(out[:, 0, 0])
